# baseline (speedup 1.0000x reference)
import functools

import jax
import jax.numpy as jnp
from jax import lax
from jax.experimental import pallas as pl
from jax.experimental.pallas import tpu as pltpu
from jax.experimental.pallas import tpu_sc as plsc

N = 10000
D = 128
E = 320000
NPAD = 10240
NC = 2
NS = 16
B = 88
G = 8
NGRP = 30
NPAIR = NGRP // 2
CH = G * NGRP
EPT = CH * B
E_PAD = NS * EPT
RPT = 624
TAIL = N - NS * RPT


def _tc_transform(x_user_p, x_item_p, W_msg_u2i, W_self_u2i, W_msg_i2u, W_self_i2u):
    BLK = 256

    def body(xu_ref, xi_ref, wm0_ref, ws0_ref, wm1_ref, ws1_ref, y_ref, b_ref):
        xu = xu_ref[...]
        xi = xi_ref[...]
        y_ref[0] = jnp.dot(xu, wm0_ref[...], preferred_element_type=jnp.float32)
        y_ref[1] = jnp.dot(xi, wm1_ref[...], preferred_element_type=jnp.float32)
        b_ref[0] = jnp.dot(xi, ws0_ref[...], preferred_element_type=jnp.float32)
        b_ref[1] = jnp.dot(xu, ws1_ref[...], preferred_element_type=jnp.float32)

    grid = (NPAD // BLK,)
    w_spec = pl.BlockSpec((D, D), lambda i: (0, 0))
    return pl.pallas_call(
        body,
        grid=grid,
        in_specs=[
            pl.BlockSpec((BLK, D), lambda i: (i, 0)),
            pl.BlockSpec((BLK, D), lambda i: (i, 0)),
            w_spec, w_spec, w_spec, w_spec,
        ],
        out_specs=[
            pl.BlockSpec((NC, BLK, D), lambda i: (0, i, 0)),
            pl.BlockSpec((NC, BLK, D), lambda i: (0, i, 0)),
        ],
        out_shape=[
            jax.ShapeDtypeStruct((NC, NPAD, D), jnp.float32),
            jax.ShapeDtypeStruct((NC, NPAD, D), jnp.float32),
        ],
    )(x_user_p, x_item_p, W_msg_u2i, W_self_u2i, W_msg_i2u, W_self_i2u)


def _sc_conv(y_flat, e_src, e_dst, base_all):
    mesh = plsc.VectorSubcoreMesh(core_axis_name="c", subcore_axis_name="s")

    @functools.partial(
        pl.kernel,
        out_type=(
            jax.ShapeDtypeStruct((N, D), jnp.float32),
            jax.ShapeDtypeStruct((N, D), jnp.float32),
        ),
        mesh=mesh,
        scratch_types=[
            pltpu.VMEM((G, B), jnp.int32),
            pltpu.VMEM((G, B), jnp.int32),
            pltpu.VMEM((G, B), jnp.int32),
            pltpu.VMEM((G, B), jnp.int32),
            pltpu.VMEM((B, D), jnp.float32),
            pltpu.VMEM((B, D), jnp.float32),
            pltpu.VMEM((B, D), jnp.float32),
            pltpu.VMEM((B, D), jnp.float32),
            pltpu.SemaphoreType.DMA,
            pltpu.SemaphoreType.DMA,
            pltpu.SemaphoreType.DMA,
            pltpu.SemaphoreType.DMA,
            pltpu.SemaphoreType.DMA,
            pltpu.SemaphoreType.DMA,
            pltpu.SemaphoreType.DMA,
            pltpu.SemaphoreType.DMA,
            pltpu.SemaphoreType.DMA,
            pltpu.SemaphoreType.DMA,
            pltpu.VMEM_SHARED((N, D), jnp.float32),
        ],
    )
    def k(y_ref, src_ref, dst_ref, base_ref, out_user, out_item,
          sidx0, sidx1, didx0, didx1, r0, r1, r2, r3,
          sg0, sg1, sg2, sg3, ss0, ss1, ss2, ss3, si0, si1, acc):
        cid = lax.axis_index("c")
        sid = lax.axis_index("s")
        row0 = pl.multiple_of(sid * RPT, 8)
        pltpu.sync_copy(base_ref.at[cid, pl.ds(row0, RPT)], acc.at[pl.ds(row0, RPT)])

        @pl.when(sid == NS - 1)
        def _():
            pltpu.sync_copy(base_ref.at[cid, pl.ds(NS * RPT, TAIL)],
                            acc.at[pl.ds(NS * RPT, TAIL)])

        plsc.subcore_barrier()

        RW = (r0, r1, r2, r3)
        SG = (sg0, sg1, sg2, sg3)
        SS = (ss0, ss1, ss2, ss3)
        SIDX = (sidx0, sidx1)
        DIDX = (didx0, didx1)
        SI = (si0, si1)

        def refill(slot, h):
            h0 = pl.multiple_of(h * G, 8)
            pltpu.async_copy(src_ref.at[cid, sid, pl.ds(h0, G)], SIDX[slot], SI[slot])
            pltpu.async_copy(dst_ref.at[cid, sid, pl.ds(h0, G)], DIDX[slot], SI[slot])

        def wait_refill(slot):
            pltpu.make_async_copy(src_ref.at[cid, sid, pl.ds(0, G)],
                                  SIDX[slot], SI[slot]).wait()
            pltpu.make_async_copy(dst_ref.at[cid, sid, pl.ds(0, G)],
                                  DIDX[slot], SI[slot]).wait()

        def gather(slot, p, b):
            pltpu.async_copy(y_ref.at[SIDX[slot].at[p]], RW[b], SG[b])

        def wait_gather(slot, p, b):
            pltpu.make_async_copy(y_ref.at[SIDX[slot].at[p]], RW[b], SG[b]).wait()

        def scatter(slot, p, b):
            pltpu.async_copy(RW[b], acc.at[DIDX[slot].at[p]], SS[b], add=True)

        def wait_scatter(b):
            pltpu.make_async_copy(RW[b], acc.at[DIDX[0].at[0]], SS[b]).wait()

        def chunk(gip, p, i, pred, first_pair):
            kk = 8 * gip + p
            b = kk % 4
            slot = gip
            if p == 2:
                if gip == 0:
                    refill(1, 2 * i + 1)
                elif first_pair:
                    refill(0, 2 * i + 2)
                else:
                    @pl.when(pred)
                    def _():
                        refill(0, 2 * i + 2)
            if not (first_pair and kk < 2):
                wait_scatter((b + 2) % 4)
            if p == 6:
                if gip == 0 or first_pair:
                    wait_refill(1 - slot)
                else:
                    @pl.when(pred)
                    def _():
                        wait_refill(1 - slot)
            if p < 6:
                gather(slot, p + 2, (b + 2) % 4)
            elif gip == 0 or first_pair:
                gather(1 - slot, p - 6, (b + 2) % 4)
            else:
                @pl.when(pred)
                def _():
                    gather(1 - slot, p - 6, (b + 2) % 4)
            wait_gather(slot, p, b)
            scatter(slot, p, b)

        refill(0, 0)
        wait_refill(0)
        gather(0, 0, 0)
        gather(0, 1, 1)
        for gip in range(2):
            for p in range(G):
                chunk(gip, p, 0, None, True)

        def pair(i, carry):
            pred = i < NPAIR - 1
            for gip in range(2):
                for p in range(G):
                    chunk(gip, p, i, pred, False)
            return carry

        lax.fori_loop(1, NPAIR, pair, 0)
        wait_scatter(2)
        wait_scatter(3)
        plsc.subcore_barrier()

        @pl.when(cid == 0)
        def _():
            pltpu.sync_copy(acc.at[pl.ds(row0, RPT)], out_item.at[pl.ds(row0, RPT)])

            @pl.when(sid == NS - 1)
            def _():
                pltpu.sync_copy(acc.at[pl.ds(NS * RPT, TAIL)],
                                out_item.at[pl.ds(NS * RPT, TAIL)])

        @pl.when(cid == 1)
        def _():
            pltpu.sync_copy(acc.at[pl.ds(row0, RPT)], out_user.at[pl.ds(row0, RPT)])

            @pl.when(sid == NS - 1)
            def _():
                pltpu.sync_copy(acc.at[pl.ds(NS * RPT, TAIL)],
                                out_user.at[pl.ds(NS * RPT, TAIL)])

    return k(y_flat, e_src, e_dst, base_all)


def _prep_edges(edge_index_u2i, edge_index_i2u):
    src0 = edge_index_u2i[0].astype(jnp.int32)
    dst0 = edge_index_u2i[1].astype(jnp.int32)
    src1 = edge_index_i2u[0].astype(jnp.int32) + NPAD
    dst1 = edge_index_i2u[1].astype(jnp.int32)
    npad = E_PAD - E
    pad0 = jnp.full((npad,), N, jnp.int32)
    pad1 = jnp.full((npad,), NPAD + N, jnp.int32)
    padd = jnp.zeros((npad,), jnp.int32)
    e_src = jnp.stack([jnp.concatenate([src0, pad0]),
                       jnp.concatenate([src1, pad1])]).reshape(NC, NS, CH, B)
    e_dst = jnp.stack([jnp.concatenate([dst0, padd]),
                       jnp.concatenate([dst1, padd])]).reshape(NC, NS, CH, B)
    return e_src, e_dst


def kernel(x_user, x_item, edge_index_u2i, edge_index_i2u,
           W_msg_u2i, W_self_u2i, W_msg_i2u, W_self_i2u):
    x_user_p = jnp.pad(x_user, ((0, NPAD - N), (0, 0)))
    x_item_p = jnp.pad(x_item, ((0, NPAD - N), (0, 0)))
    e_src, e_dst = _prep_edges(edge_index_u2i, edge_index_i2u)
    y_all, base_all = _tc_transform(x_user_p, x_item_p,
                                    W_msg_u2i, W_self_u2i, W_msg_i2u, W_self_i2u)
    y_flat = y_all.reshape(NC * NPAD, D)
    out_user, out_item = _sc_conv(y_flat, e_src, e_dst, base_all)
    return (out_user, out_item)


# speedup vs baseline: 2.0725x; 2.0725x over previous
import functools

import jax
import jax.numpy as jnp
from jax import lax
from jax.experimental import pallas as pl
from jax.experimental.pallas import tpu as pltpu
from jax.experimental.pallas import tpu_sc as plsc

N = 10000
D = 128
E = 320000
NPAD = 10240
NC = 2
NS = 16
B = 128
G = 8
NGRP = 20
NPAIR = NGRP // 2
CH = G * NGRP
EPT = CH * B
E_PAD = NS * EPT
RPT = 624
TAIL = N - NS * RPT


def _tc_transform(x_user_p, x_item_p, W_msg_u2i, W_self_u2i, W_msg_i2u, W_self_i2u):
    BLK = 256

    def body(xu_ref, xi_ref, wm0_ref, ws0_ref, wm1_ref, ws1_ref, y_ref, b_ref):
        xu = xu_ref[...]
        xi = xi_ref[...]
        y_ref[0] = jnp.dot(xu, wm0_ref[...], preferred_element_type=jnp.float32)
        y_ref[1] = jnp.dot(xi, wm1_ref[...], preferred_element_type=jnp.float32)
        b_ref[0] = jnp.dot(xi, ws0_ref[...], preferred_element_type=jnp.float32)
        b_ref[1] = jnp.dot(xu, ws1_ref[...], preferred_element_type=jnp.float32)

    grid = (NPAD // BLK,)
    w_spec = pl.BlockSpec((D, D), lambda i: (0, 0))
    return pl.pallas_call(
        body,
        grid=grid,
        in_specs=[
            pl.BlockSpec((BLK, D), lambda i: (i, 0)),
            pl.BlockSpec((BLK, D), lambda i: (i, 0)),
            w_spec, w_spec, w_spec, w_spec,
        ],
        out_specs=[
            pl.BlockSpec((NC, BLK, D), lambda i: (0, i, 0)),
            pl.BlockSpec((NC, BLK, D), lambda i: (0, i, 0)),
        ],
        out_shape=[
            jax.ShapeDtypeStruct((NC, NPAD, D), jnp.float32),
            jax.ShapeDtypeStruct((NC, NPAD, D), jnp.float32),
        ],
    )(x_user_p, x_item_p, W_msg_u2i, W_self_u2i, W_msg_i2u, W_self_i2u)


def _sc_conv(y_flat, e_src, e_dst, base_all):
    mesh = plsc.VectorSubcoreMesh(core_axis_name="c", subcore_axis_name="s")

    @functools.partial(
        pl.kernel,
        out_type=(
            jax.ShapeDtypeStruct((N, D), jnp.float32),
            jax.ShapeDtypeStruct((N, D), jnp.float32),
        ),
        mesh=mesh,
        scratch_types=[
            pltpu.VMEM((G, B), jnp.int32),
            pltpu.VMEM((G, B), jnp.int32),
            pltpu.VMEM((G, B), jnp.int32),
            pltpu.VMEM((G, B), jnp.int32),
            pltpu.VMEM((B, D), jnp.float32),
            pltpu.VMEM((B, D), jnp.float32),
            pltpu.SemaphoreType.DMA,
            pltpu.SemaphoreType.DMA,
            pltpu.SemaphoreType.DMA,
            pltpu.SemaphoreType.DMA,
            pltpu.VMEM_SHARED((N, D), jnp.float32),
        ],
    )
    def k(y_ref, src_ref, dst_ref, base_ref, out_user, out_item,
          sidx0, sidx1, didx0, didx1, r0, r1,
          sg0, sg1, si0, si1, acc):
        cid = lax.axis_index("c")
        sid = lax.axis_index("s")
        row0 = pl.multiple_of(sid * RPT, 8)
        pltpu.sync_copy(base_ref.at[cid, pl.ds(row0, RPT)], acc.at[pl.ds(row0, RPT)])

        @pl.when(sid == NS - 1)
        def _():
            pltpu.sync_copy(base_ref.at[cid, pl.ds(NS * RPT, TAIL)],
                            acc.at[pl.ds(NS * RPT, TAIL)])

        plsc.subcore_barrier()

        RW = (r0, r1)
        SG = (sg0, sg1)
        SIDX = (sidx0, sidx1)
        DIDX = (didx0, didx1)
        SI = (si0, si1)

        def refill(slot, h):
            h0 = pl.multiple_of(h * G, 8)
            pltpu.async_copy(src_ref.at[cid, sid, pl.ds(h0, G)], SIDX[slot], SI[slot])
            pltpu.async_copy(dst_ref.at[cid, sid, pl.ds(h0, G)], DIDX[slot], SI[slot])

        def wait_refill(slot):
            pltpu.make_async_copy(src_ref.at[cid, sid, pl.ds(0, G)],
                                  SIDX[slot], SI[slot]).wait()
            pltpu.make_async_copy(dst_ref.at[cid, sid, pl.ds(0, G)],
                                  DIDX[slot], SI[slot]).wait()

        def gather(slot, p, b):
            pltpu.async_copy(y_ref.at[SIDX[slot].at[p]], RW[b], SG[b])

        def wait_gather(slot, p, b):
            pltpu.make_async_copy(y_ref.at[SIDX[slot].at[p]], RW[b], SG[b]).wait()

        def chunk(gip, p, i, pred, first_pair):
            kk = 8 * gip + p
            b = kk % 2
            slot = gip
            if p == 2:
                if gip == 0:
                    refill(1, 2 * i + 1)
                elif first_pair:
                    refill(0, 2 * i + 2)
                else:
                    @pl.when(pred)
                    def _():
                        refill(0, 2 * i + 2)
            if p == 6:
                if gip == 0 or first_pair:
                    wait_refill(1 - slot)
                else:
                    @pl.when(pred)
                    def _():
                        wait_refill(1 - slot)
            wait_gather(slot, p, b)
            pltpu.sync_copy(RW[b], acc.at[DIDX[slot].at[p]], add=True)
            if p < 6:
                gather(slot, p + 2, b)
            elif gip == 0 or first_pair:
                gather(1 - slot, p - 6, b)
            else:
                @pl.when(pred)
                def _():
                    gather(1 - slot, p - 6, b)

        refill(0, 0)
        wait_refill(0)
        gather(0, 0, 0)
        gather(0, 1, 1)
        for gip in range(2):
            for p in range(G):
                chunk(gip, p, 0, None, True)

        def pair(i, carry):
            pred = i < NPAIR - 1
            for gip in range(2):
                for p in range(G):
                    chunk(gip, p, i, pred, False)
            return carry

        lax.fori_loop(1, NPAIR, pair, 0)
        plsc.subcore_barrier()

        @pl.when(cid == 0)
        def _():
            pltpu.sync_copy(acc.at[pl.ds(row0, RPT)], out_item.at[pl.ds(row0, RPT)])

            @pl.when(sid == NS - 1)
            def _():
                pltpu.sync_copy(acc.at[pl.ds(NS * RPT, TAIL)],
                                out_item.at[pl.ds(NS * RPT, TAIL)])

        @pl.when(cid == 1)
        def _():
            pltpu.sync_copy(acc.at[pl.ds(row0, RPT)], out_user.at[pl.ds(row0, RPT)])

            @pl.when(sid == NS - 1)
            def _():
                pltpu.sync_copy(acc.at[pl.ds(NS * RPT, TAIL)],
                                out_user.at[pl.ds(NS * RPT, TAIL)])

    return k(y_flat, e_src, e_dst, base_all)


def _prep_edges(edge_index_u2i, edge_index_i2u):
    src0 = edge_index_u2i[0].astype(jnp.int32)
    dst0 = edge_index_u2i[1].astype(jnp.int32)
    src1 = edge_index_i2u[0].astype(jnp.int32) + NPAD
    dst1 = edge_index_i2u[1].astype(jnp.int32)
    npad = E_PAD - E
    pad0 = jnp.full((npad,), N, jnp.int32)
    pad1 = jnp.full((npad,), NPAD + N, jnp.int32)
    padd = jnp.zeros((npad,), jnp.int32)
    e_src = jnp.stack([jnp.concatenate([src0, pad0]),
                       jnp.concatenate([src1, pad1])]).reshape(NC, NS, CH, B)
    e_dst = jnp.stack([jnp.concatenate([dst0, padd]),
                       jnp.concatenate([dst1, padd])]).reshape(NC, NS, CH, B)
    return e_src, e_dst


def kernel(x_user, x_item, edge_index_u2i, edge_index_i2u,
           W_msg_u2i, W_self_u2i, W_msg_i2u, W_self_i2u):
    x_user_p = jnp.pad(x_user, ((0, NPAD - N), (0, 0)))
    x_item_p = jnp.pad(x_item, ((0, NPAD - N), (0, 0)))
    e_src, e_dst = _prep_edges(edge_index_u2i, edge_index_i2u)
    y_all, base_all = _tc_transform(x_user_p, x_item_p,
                                    W_msg_u2i, W_self_u2i, W_msg_i2u, W_self_i2u)
    y_flat = y_all.reshape(NC * NPAD, D)
    out_user, out_item = _sc_conv(y_flat, e_src, e_dst, base_all)
    return (out_user, out_item)
